# baseline (speedup 1.0000x reference)
import itertools

import jax
import jax.numpy as jnp
import numpy as np
from jax import lax
from jax.experimental import pallas as pl
from jax.experimental.pallas import tpu as pltpu
from jax.experimental.pallas import tpu_sc as plsc

_B = 131072
_N_LEVEL = 16
_N_ENTRIES = 1048576
_MASK = _N_ENTRIES - 1
_P2 = np.int32(2654435761 - (1 << 32))
_P3 = np.int32(805459861)

_NW = 32
_SUB = 2048
_NSUB = _B // (_NW * _SUB)
_NG = _SUB // 16


def _enc_body(xt_hbm, tbl_hbm, out_hbm, sem, *bufs):
    x_vs = bufs[0:3]
    w_vs = bufs[3:11]
    f_vs = bufs[11:13]
    idx_vs = bufs[13:21]
    rows_vs = bufs[21:29]
    wid = lax.axis_index("s") * 2 + lax.axis_index("c")

    for sub in range(_NSUB):
        base = wid * (_NSUB * _SUB) + sub * _SUB
        for d in range(3):
            pltpu.sync_copy(xt_hbm.at[pl.ds(d * _B + base, _SUB)], x_vs[d])

        def level_body(l, res, base=base):
            loff = l * _N_ENTRIES

            def p1(i, carry):
                sl = pl.ds(i * 16, 16)
                pos0 = x_vs[0][sl] * res
                pos1 = x_vs[1][sl] * res
                pos2 = x_vs[2][sl] * res
                ci0 = pos0.astype(jnp.int32)
                ci1 = pos1.astype(jnp.int32)
                ci2 = pos2.astype(jnp.int32)
                fr0 = pos0 - ci0.astype(jnp.float32)
                fr1 = pos1 - ci1.astype(jnp.float32)
                fr2 = pos2 - ci2.astype(jnp.float32)
                g0 = 1.0 - fr0
                g1 = 1.0 - fr1
                g2 = 1.0 - fr2
                hy0 = ci1 * _P2
                hz0 = ci2 * _P3
                hx1 = ci0 + 1
                hy1 = hy0 + _P2
                hz1 = hz0 + _P3
                e = (ci0 ^ hy0, ci0 ^ hy1, hx1 ^ hy0, hx1 ^ hy1)
                wxy = (g0 * g1, g0 * fr1, fr0 * g1, fr0 * fr1)
                for cidx, (ox, oy, oz) in enumerate(
                        itertools.product((0, 1), repeat=3)):
                    h = e[ox * 2 + oy] ^ (hz1 if oz else hz0)
                    idx_vs[cidx][sl] = (h & _MASK) + loff
                    w_vs[cidx][sl] = wxy[ox * 2 + oy] * (fr2 if oz else g2)
                return carry

            lax.fori_loop(0, _NG, p1, 0)

            cps = [pltpu.async_copy(tbl_hbm.at[idx_vs[k]], rows_vs[k], sem)
                   for k in range(8)]
            for cp in cps:
                cp.wait()

            def p2(j, carry):
                sl = pl.ds(j * 16, 16)
                f0 = jnp.zeros((16,), jnp.float32)
                f1 = jnp.zeros((16,), jnp.float32)
                for c in range(8):
                    w = w_vs[c][sl]
                    r = rows_vs[c][sl]
                    r0 = lax.bitcast_convert_type(r << 16, jnp.float32)
                    r1 = lax.bitcast_convert_type(r & np.int32(-65536), jnp.float32)
                    f0 = f0 + w * r0
                    f1 = f1 + w * r1
                f_vs[0][sl] = f0
                f_vs[1][sl] = f1
                return carry

            lax.fori_loop(0, _NG, p2, 0)

            row = 2 * l * _B + base
            pltpu.sync_copy(f_vs[0], out_hbm.at[pl.ds(row, _SUB)])
            pltpu.sync_copy(f_vs[1], out_hbm.at[pl.ds(row + _B, _SUB)])
            return res * 1.5

        lax.fori_loop(0, _N_LEVEL, level_body, jnp.float32(16.0))


def _encode_sc(xt, tbl):
    mesh = plsc.VectorSubcoreMesh(core_axis_name="c", subcore_axis_name="s")
    return pl.kernel(
        _enc_body,
        out_type=jax.ShapeDtypeStruct((2 * _N_LEVEL * _B,), jnp.float32),
        mesh=mesh,
        scratch_types=[pltpu.SemaphoreType.DMA]
          + [pltpu.VMEM((_SUB,), jnp.float32) for _ in range(13)]
          + [pltpu.VMEM((_SUB,), jnp.int32) for _ in range(8)]
          + [pltpu.VMEM((_SUB,), jnp.int32) for _ in range(8)],
    )(xt, tbl)


def _mlp_body(ft_ref, w1, b1, w2, b2, w3, b3, w4, b4, out_ref):
    dn = (((0,), (0,)), ((), ()))
    ft = ft_ref[...]
    h = jnp.maximum(
        lax.dot_general(w1[...], ft, dn, preferred_element_type=jnp.float32)
        + b1[...], 0.0)
    h = jnp.maximum(
        lax.dot_general(w2[...], h, dn, preferred_element_type=jnp.float32)
        + b2[...], 0.0)
    h = jnp.maximum(
        lax.dot_general(w3[...], h, dn, preferred_element_type=jnp.float32)
        + b3[...], 0.0)
    out_ref[...] = (
        lax.dot_general(h, w4[...], dn, preferred_element_type=jnp.float32)
        + b4[...])


def _mlp(featsT, W1, b1, W2, b2, W3, b3, W4, b4):
    BT = 8192
    grid = (_B // BT,)
    full = lambda shape: pl.BlockSpec(shape, lambda i: (0, 0))
    return pl.pallas_call(
        _mlp_body,
        grid=grid,
        in_specs=[
            pl.BlockSpec((32, BT), lambda i: (0, i)),
            full((32, 64)), full((64, 1)),
            full((64, 64)), full((64, 1)),
            full((64, 64)), full((64, 1)),
            full((64, 3)), full((1, 3)),
        ],
        out_specs=pl.BlockSpec((BT, 3), lambda i: (i, 0)),
        out_shape=jax.ShapeDtypeStruct((_B, 3), jnp.float32),
    )(featsT, W1, b1.reshape(64, 1), W2, b2.reshape(64, 1),
      W3, b3.reshape(64, 1), W4, b4.reshape(1, 3))


def kernel(x, tables, W1, b1, W2, b2, W3, b3, W4, b4):
    t4 = tables.reshape(_N_LEVEL, _N_ENTRIES // 128, 128, 2)
    u = lax.bitcast_convert_type(t4, jnp.uint32)
    u0 = u[..., 0]
    u1 = u[..., 1]
    r0 = (u0 + jnp.uint32(0x7FFF) + ((u0 >> 16) & 1)) >> 16
    r1 = ((u1 + jnp.uint32(0x7FFF) + ((u1 >> 16) & 1)) >> 16) << 16
    tbl = lax.bitcast_convert_type(r0 | r1, jnp.int32).reshape(
        _N_LEVEL * _N_ENTRIES)
    featsT = _encode_sc(x.T.reshape(3 * _B), tbl).reshape(2 * _N_LEVEL, _B)
    return _mlp(featsT, W1, b1, W2, b2, W3, b3, W4, b4)
    return _mlp(featsT, W1, b1, W2, b2, W3, b3, W4, b4)


# speedup vs baseline: 1.0496x; 1.0496x over previous
import itertools

import jax
import jax.numpy as jnp
import numpy as np
from jax import lax
from jax.experimental import pallas as pl
from jax.experimental.pallas import tpu as pltpu
from jax.experimental.pallas import tpu_sc as plsc

_B = 131072
_N_LEVEL = 16
_N_ENTRIES = 1048576
_MASK = _N_ENTRIES - 1
_P2 = np.int32(2654435761 - (1 << 32))
_P3 = np.int32(805459861)

_NW = 32
_SUB = 2048
_NSUB = _B // (_NW * _SUB)
_NG = _SUB // 16


def _enc_body(xt_hbm, tbl_hbm, out_hbm, sem, *bufs):
    x_vs = bufs[0:3]
    w_vs = bufs[3:11]
    f_vs = bufs[11:13]
    idx_vs = bufs[13:21]
    rows_vs = bufs[21:29]
    wid = lax.axis_index("s") * 2 + lax.axis_index("c")

    for sub in range(_NSUB):
        base = wid * (_NSUB * _SUB) + sub * _SUB
        for d in range(3):
            pltpu.sync_copy(xt_hbm.at[pl.ds(d * _B + base, _SUB)], x_vs[d])

        def level_body(l, res, base=base):
            loff = l * _N_ENTRIES

            def p1(i, carry):
                sl = pl.ds(i * 16, 16)
                pos0 = x_vs[0][sl] * res
                pos1 = x_vs[1][sl] * res
                pos2 = x_vs[2][sl] * res
                ci0 = pos0.astype(jnp.int32)
                ci1 = pos1.astype(jnp.int32)
                ci2 = pos2.astype(jnp.int32)
                fr0 = pos0 - ci0.astype(jnp.float32)
                fr1 = pos1 - ci1.astype(jnp.float32)
                fr2 = pos2 - ci2.astype(jnp.float32)
                g0 = 1.0 - fr0
                g1 = 1.0 - fr1
                g2 = 1.0 - fr2
                hy0 = ci1 * _P2
                hz0 = ci2 * _P3
                hx1 = ci0 + 1
                hy1 = hy0 + _P2
                hz1 = hz0 + _P3
                e = (ci0 ^ hy0, ci0 ^ hy1, hx1 ^ hy0, hx1 ^ hy1)
                wxy = (g0 * g1, g0 * fr1, fr0 * g1, fr0 * fr1)
                for cidx, (ox, oy, oz) in enumerate(
                        itertools.product((0, 1), repeat=3)):
                    h = e[ox * 2 + oy] ^ (hz1 if oz else hz0)
                    idx_vs[cidx][sl] = (h & _MASK) + loff
                    w_vs[cidx][sl] = wxy[ox * 2 + oy] * (fr2 if oz else g2)
                return carry

            lax.fori_loop(0, _NG, p1, 0)

            cps = [pltpu.async_copy(tbl_hbm.at[idx_vs[k]], rows_vs[k], sem)
                   for k in range(8)]
            for cp in cps:
                cp.wait()

            def p2(j, carry):
                sl = pl.ds(j * 16, 16)
                f0 = jnp.zeros((16,), jnp.float32)
                f1 = jnp.zeros((16,), jnp.float32)
                for c in range(8):
                    w = w_vs[c][sl]
                    r = rows_vs[c][sl]
                    r0 = lax.bitcast_convert_type(r << 16, jnp.float32)
                    r1 = lax.bitcast_convert_type(r & np.int32(-65536), jnp.float32)
                    f0 = f0 + w * r0
                    f1 = f1 + w * r1
                f_vs[0][sl] = f0
                f_vs[1][sl] = f1
                return carry

            lax.fori_loop(0, _NG, p2, 0)

            row = 2 * l * _B + base
            pltpu.sync_copy(f_vs[0], out_hbm.at[pl.ds(row, _SUB)])
            pltpu.sync_copy(f_vs[1], out_hbm.at[pl.ds(row + _B, _SUB)])
            return res * 1.5

        lax.fori_loop(0, _N_LEVEL, level_body, jnp.float32(16.0))


def _encode_sc(xt, tbl):
    mesh = plsc.VectorSubcoreMesh(core_axis_name="c", subcore_axis_name="s")
    return pl.kernel(
        _enc_body,
        out_type=jax.ShapeDtypeStruct((2 * _N_LEVEL * _B,), jnp.float32),
        mesh=mesh,
        scratch_types=[pltpu.SemaphoreType.DMA]
          + [pltpu.VMEM((_SUB,), jnp.float32) for _ in range(13)]
          + [pltpu.VMEM((_SUB,), jnp.int32) for _ in range(8)]
          + [pltpu.VMEM((_SUB,), jnp.int32) for _ in range(8)],
    )(xt, tbl)


def _mlp_body(ft_ref, w1, b1, w2, b2, w3, b3, w4, b4, out_ref):
    dn = (((0,), (0,)), ((), ()))
    ft = ft_ref[...]
    h = jnp.maximum(
        lax.dot_general(w1[...], ft, dn, preferred_element_type=jnp.float32)
        + b1[...], 0.0)
    h = jnp.maximum(
        lax.dot_general(w2[...], h, dn, preferred_element_type=jnp.float32)
        + b2[...], 0.0)
    h = jnp.maximum(
        lax.dot_general(w3[...], h, dn, preferred_element_type=jnp.float32)
        + b3[...], 0.0)
    out_ref[...] = (
        lax.dot_general(h, w4[...], dn, preferred_element_type=jnp.float32)
        + b4[...])


def _mlp(featsT, W1, b1, W2, b2, W3, b3, W4, b4):
    BT = 8192
    grid = (_B // BT,)
    full = lambda shape: pl.BlockSpec(shape, lambda i: (0, 0))
    return pl.pallas_call(
        _mlp_body,
        grid=grid,
        in_specs=[
            pl.BlockSpec((32, BT), lambda i: (0, i)),
            full((32, 64)), full((64, 1)),
            full((64, 64)), full((64, 1)),
            full((64, 64)), full((64, 1)),
            full((64, 3)), full((1, 3)),
        ],
        out_specs=pl.BlockSpec((BT, 3), lambda i: (i, 0)),
        out_shape=jax.ShapeDtypeStruct((_B, 3), jnp.float32),
    )(featsT, W1, b1.reshape(64, 1), W2, b2.reshape(64, 1),
      W3, b3.reshape(64, 1), W4, b4.reshape(1, 3))


def kernel(x, tables, W1, b1, W2, b2, W3, b3, W4, b4):
    t4 = tables.reshape(_N_LEVEL, _N_ENTRIES // 128, 128, 2)
    tu = lax.bitcast_convert_type(t4.astype(jnp.bfloat16), jnp.uint16)
    packed = (tu[..., 0].astype(jnp.uint32)
              | (tu[..., 1].astype(jnp.uint32) << 16))
    tbl = lax.bitcast_convert_type(packed, jnp.int32).reshape(
        _N_LEVEL * _N_ENTRIES)
    featsT = _encode_sc(x.T.reshape(3 * _B), tbl).reshape(2 * _N_LEVEL, _B)
    return _mlp(featsT, W1, b1, W2, b2, W3, b3, W4, b4)
    return _mlp(featsT, W1, b1, W2, b2, W3, b3, W4, b4)


# speedup vs baseline: 1.1924x; 1.1361x over previous
import itertools

import jax
import jax.numpy as jnp
import numpy as np
from jax import lax
from jax.experimental import pallas as pl
from jax.experimental.pallas import tpu as pltpu
from jax.experimental.pallas import tpu_sc as plsc

_B = 131072
_N_LEVEL = 16
_N_ENTRIES = 1048576
_MASK = _N_ENTRIES - 1
_P2 = np.int32(2654435761 - (1 << 32))
_P3 = np.int32(805459861)

_NW = 32
_SUB = 2048
_NSUB = _B // (_NW * _SUB)
_NG = _SUB // 16


def _enc_body(xt_hbm, tbl_hbm, out_hbm, semA, semB, semOA, semOB, *bufs):
    x_vs = bufs[0:3]
    wA, wB = bufs[3:11], bufs[11:19]
    fA, fB = bufs[19:21], bufs[21:23]
    idxA, idxB = bufs[23:31], bufs[31:39]
    rowsA, rowsB = bufs[39:47], bufs[47:55]
    wid = lax.axis_index("s") * 2 + lax.axis_index("c")

    def p1(l, res, idxs, ws):
        loff = l * _N_ENTRIES

        def body(i, carry):
            sl = pl.ds(i * 16, 16)
            pos0 = x_vs[0][sl] * res
            pos1 = x_vs[1][sl] * res
            pos2 = x_vs[2][sl] * res
            ci0 = pos0.astype(jnp.int32)
            ci1 = pos1.astype(jnp.int32)
            ci2 = pos2.astype(jnp.int32)
            fr0 = pos0 - ci0.astype(jnp.float32)
            fr1 = pos1 - ci1.astype(jnp.float32)
            fr2 = pos2 - ci2.astype(jnp.float32)
            g0 = 1.0 - fr0
            g1 = 1.0 - fr1
            g2 = 1.0 - fr2
            hy0 = ci1 * _P2
            hz0 = ci2 * _P3
            hx1 = ci0 + 1
            hy1 = hy0 + _P2
            hz1 = hz0 + _P3
            e = (ci0 ^ hy0, ci0 ^ hy1, hx1 ^ hy0, hx1 ^ hy1)
            wxy = (g0 * g1, g0 * fr1, fr0 * g1, fr0 * fr1)
            for cidx, (ox, oy, oz) in enumerate(
                    itertools.product((0, 1), repeat=3)):
                h = e[ox * 2 + oy] ^ (hz1 if oz else hz0)
                idxs[cidx][sl] = (h & _MASK) + loff
                ws[cidx][sl] = wxy[ox * 2 + oy] * (fr2 if oz else g2)
            return carry

        lax.fori_loop(0, _NG, body, 0)

    def fire(idxs, rows, sem):
        for c in range(8):
            pltpu.async_copy(tbl_hbm.at[idxs[c]], rows[c], sem)

    def drain(idxs, rows, sem):
        for c in range(8):
            pltpu.make_async_copy(tbl_hbm.at[idxs[c]], rows[c], sem).wait()

    def p2(l, base, rows, ws, fs, semO):
        def body(j, carry):
            sl = pl.ds(j * 16, 16)
            f0 = jnp.zeros((16,), jnp.float32)
            f1 = jnp.zeros((16,), jnp.float32)
            for c in range(8):
                w = ws[c][sl]
                r = rows[c][sl]
                r0 = lax.bitcast_convert_type(r << 16, jnp.float32)
                r1 = lax.bitcast_convert_type(r & np.int32(-65536), jnp.float32)
                f0 = f0 + w * r0
                f1 = f1 + w * r1
            fs[0][sl] = f0
            fs[1][sl] = f1
            return carry

        lax.fori_loop(0, _NG, body, 0)
        row = 2 * l * _B + base
        pltpu.async_copy(fs[0], out_hbm.at[pl.ds(row, _SUB)], semO)
        pltpu.async_copy(fs[1], out_hbm.at[pl.ds(row + _B, _SUB)], semO)

    def drain_out(fs, semO, base):
        pltpu.make_async_copy(fs[0], out_hbm.at[pl.ds(base, _SUB)], semO).wait()
        pltpu.make_async_copy(fs[1], out_hbm.at[pl.ds(base, _SUB)], semO).wait()

    for sub in range(_NSUB):
        base = wid * (_NSUB * _SUB) + sub * _SUB
        for d in range(3):
            pltpu.sync_copy(xt_hbm.at[pl.ds(d * _B + base, _SUB)], x_vs[d])

        p1(0, jnp.float32(16.0), idxA, wA)
        fire(idxA, rowsA, semA)

        def pair_body(k, res, base=base):
            a = 2 * k
            b = a + 1
            p1(b, res * 1.5, idxB, wB)
            drain(idxA, rowsA, semA)
            fire(idxB, rowsB, semB)

            @pl.when(k > 0)
            def _():
                drain_out(fA, semOA, base)

            p2(a, base, rowsA, wA, fA, semOA)

            @pl.when(k < (_N_LEVEL // 2 - 1))
            def _():
                p1(a + 2, res * 2.25, idxA, wA)

            drain(idxB, rowsB, semB)

            @pl.when(k < (_N_LEVEL // 2 - 1))
            def _():
                fire(idxA, rowsA, semA)

            @pl.when(k > 0)
            def _():
                drain_out(fB, semOB, base)

            p2(b, base, rowsB, wB, fB, semOB)
            return res * 2.25

        lax.fori_loop(0, _N_LEVEL // 2, pair_body, jnp.float32(16.0))
        drain_out(fA, semOA, base)
        drain_out(fB, semOB, base)


def _encode_sc(xt, tbl):
    mesh = plsc.VectorSubcoreMesh(core_axis_name="c", subcore_axis_name="s")
    return pl.kernel(
        _enc_body,
        out_type=jax.ShapeDtypeStruct((2 * _N_LEVEL * _B,), jnp.float32),
        mesh=mesh,
        scratch_types=[pltpu.SemaphoreType.DMA for _ in range(4)]
          + [pltpu.VMEM((_SUB,), jnp.float32) for _ in range(23)]
          + [pltpu.VMEM((_SUB,), jnp.int32) for _ in range(32)],
    )(xt, tbl)


def _mlp_body(ft_ref, w1, b1, w2, b2, w3, b3, w4, b4, out_ref):
    dn = (((0,), (0,)), ((), ()))
    ft = ft_ref[...]
    h = jnp.maximum(
        lax.dot_general(w1[...], ft, dn, preferred_element_type=jnp.float32)
        + b1[...], 0.0)
    h = jnp.maximum(
        lax.dot_general(w2[...], h, dn, preferred_element_type=jnp.float32)
        + b2[...], 0.0)
    h = jnp.maximum(
        lax.dot_general(w3[...], h, dn, preferred_element_type=jnp.float32)
        + b3[...], 0.0)
    out_ref[...] = (
        lax.dot_general(h, w4[...], dn, preferred_element_type=jnp.float32)
        + b4[...])


def _mlp(featsT, W1, b1, W2, b2, W3, b3, W4, b4):
    BT = 8192
    grid = (_B // BT,)
    full = lambda shape: pl.BlockSpec(shape, lambda i: (0, 0))
    return pl.pallas_call(
        _mlp_body,
        grid=grid,
        in_specs=[
            pl.BlockSpec((32, BT), lambda i: (0, i)),
            full((32, 64)), full((64, 1)),
            full((64, 64)), full((64, 1)),
            full((64, 64)), full((64, 1)),
            full((64, 3)), full((1, 3)),
        ],
        out_specs=pl.BlockSpec((BT, 3), lambda i: (i, 0)),
        out_shape=jax.ShapeDtypeStruct((_B, 3), jnp.float32),
    )(featsT, W1, b1.reshape(64, 1), W2, b2.reshape(64, 1),
      W3, b3.reshape(64, 1), W4, b4.reshape(1, 3))


def kernel(x, tables, W1, b1, W2, b2, W3, b3, W4, b4):
    tu = lax.bitcast_convert_type(tables.astype(jnp.bfloat16), jnp.uint16)
    packed = (tu[..., 0].astype(jnp.uint32)
              | (tu[..., 1].astype(jnp.uint32) << 16))
    tbl = lax.bitcast_convert_type(packed, jnp.int32).reshape(
        _N_LEVEL * _N_ENTRIES)
    featsT = _encode_sc(x.T.reshape(3 * _B), tbl).reshape(2 * _N_LEVEL, _B)
    return _mlp(featsT, W1, b1, W2, b2, W3, b3, W4, b4)
    return _mlp(featsT, W1, b1, W2, b2, W3, b3, W4, b4)


# speedup vs baseline: 1.2927x; 1.0841x over previous
import itertools

import jax
import jax.numpy as jnp
import numpy as np
from jax import lax
from jax.experimental import pallas as pl
from jax.experimental.pallas import tpu as pltpu
from jax.experimental.pallas import tpu_sc as plsc

_B = 131072
_N_LEVEL = 16
_N_ENTRIES = 1048576
_MASK = _N_ENTRIES - 1
_P2 = np.int32(2654435761 - (1 << 32))
_P3 = np.int32(805459861)

_NW = 32
_SUB = 2048
_NSUB = _B // (_NW * _SUB)
_NG = _SUB // 16


def _enc_body(xt_hbm, tbl_hbm, out_hbm, semA, semB, semOA, semOB, *bufs):
    x_vs = bufs[0:3]
    wA, wB = bufs[3:11], bufs[11:19]
    fA, fB = bufs[19:21], bufs[21:23]
    idxA, idxB = bufs[23:31], bufs[31:39]
    rowsA, rowsB = bufs[39:47], bufs[47:55]
    wid = lax.axis_index("s") * 2 + lax.axis_index("c")

    def p1(l, res, idxs, ws):
        loff = l * _N_ENTRIES

        def body(i, carry):
            sl = pl.ds(i * 16, 16)
            pos0 = x_vs[0][sl] * res
            pos1 = x_vs[1][sl] * res
            pos2 = x_vs[2][sl] * res
            ci0 = pos0.astype(jnp.int32)
            ci1 = pos1.astype(jnp.int32)
            ci2 = pos2.astype(jnp.int32)
            fr0 = pos0 - ci0.astype(jnp.float32)
            fr1 = pos1 - ci1.astype(jnp.float32)
            fr2 = pos2 - ci2.astype(jnp.float32)
            g0 = 1.0 - fr0
            g1 = 1.0 - fr1
            g2 = 1.0 - fr2
            hy0 = ci1 * _P2
            hz0 = ci2 * _P3
            hx1 = ci0 + 1
            hy1 = hy0 + _P2
            hz1 = hz0 + _P3
            e = (ci0 ^ hy0, ci0 ^ hy1, hx1 ^ hy0, hx1 ^ hy1)
            wxy = (g0 * g1, g0 * fr1, fr0 * g1, fr0 * fr1)
            for cidx, (ox, oy, oz) in enumerate(
                    itertools.product((0, 1), repeat=3)):
                h = e[ox * 2 + oy] ^ (hz1 if oz else hz0)
                idxs[cidx][sl] = (h & _MASK) + loff
                ws[cidx][sl] = wxy[ox * 2 + oy] * (fr2 if oz else g2)
            return carry

        lax.fori_loop(0, _NG, body, 0)

    def fire(idxs, rows, sem):
        for c in range(8):
            pltpu.async_copy(tbl_hbm.at[idxs[c]], rows[c], sem)

    def drain(idxs, rows, sem):
        for c in range(8):
            pltpu.make_async_copy(tbl_hbm.at[idxs[c]], rows[c], sem).wait()

    def p2(l, base, rows, ws, fs, semO):
        def body(j, carry):
            sl = pl.ds(j * 16, 16)
            f0 = jnp.zeros((16,), jnp.float32)
            f1 = jnp.zeros((16,), jnp.float32)
            for c in range(8):
                w = ws[c][sl]
                r = rows[c][sl]
                r0 = lax.bitcast_convert_type(r << 16, jnp.float32)
                r1 = lax.bitcast_convert_type(r & np.int32(-65536), jnp.float32)
                f0 = f0 + w * r0
                f1 = f1 + w * r1
            fs[0][sl] = f0
            fs[1][sl] = f1
            return carry

        lax.fori_loop(0, _NG, body, 0)
        row = 2 * l * _B + base
        pltpu.async_copy(fs[0], out_hbm.at[pl.ds(row, _SUB)], semO)
        pltpu.async_copy(fs[1], out_hbm.at[pl.ds(row + _B, _SUB)], semO)

    def drain_out(fs, semO, base):
        pltpu.make_async_copy(fs[0], out_hbm.at[pl.ds(base, _SUB)], semO).wait()
        pltpu.make_async_copy(fs[1], out_hbm.at[pl.ds(base, _SUB)], semO).wait()

    for sub in range(_NSUB):
        base = wid * (_NSUB * _SUB) + sub * _SUB
        for d in range(3):
            pltpu.sync_copy(xt_hbm.at[pl.ds(d * _B + base, _SUB)], x_vs[d])

        p1(0, jnp.float32(16.0), idxA, wA)
        fire(idxA, rowsA, semA)

        def pair_body(k, res, base=base):
            a = 2 * k
            b = a + 1
            p1(b, res * 1.5, idxB, wB)
            drain(idxA, rowsA, semA)
            fire(idxB, rowsB, semB)

            @pl.when(k > 0)
            def _():
                drain_out(fA, semOA, base)

            p2(a, base, rowsA, wA, fA, semOA)

            @pl.when(k < (_N_LEVEL // 2 - 1))
            def _():
                p1(a + 2, res * 2.25, idxA, wA)

            drain(idxB, rowsB, semB)

            @pl.when(k < (_N_LEVEL // 2 - 1))
            def _():
                fire(idxA, rowsA, semA)

            @pl.when(k > 0)
            def _():
                drain_out(fB, semOB, base)

            p2(b, base, rowsB, wB, fB, semOB)
            return res * 2.25

        lax.fori_loop(0, _N_LEVEL // 2, pair_body, jnp.float32(16.0))
        drain_out(fA, semOA, base)
        drain_out(fB, semOB, base)


def _encode_sc(xt, tbl):
    mesh = plsc.VectorSubcoreMesh(core_axis_name="c", subcore_axis_name="s")
    return pl.kernel(
        _enc_body,
        out_type=jax.ShapeDtypeStruct((2 * _N_LEVEL * _B,), jnp.float32),
        mesh=mesh,
        scratch_types=[pltpu.SemaphoreType.DMA for _ in range(4)]
          + [pltpu.VMEM((_SUB,), jnp.float32) for _ in range(23)]
          + [pltpu.VMEM((_SUB,), jnp.int32) for _ in range(32)],
    )(xt, tbl)


def _pack_body(t_ref, o_ref):
    t = t_ref[...]
    t3 = t.reshape(t.shape[0] // 2, 2, 128)
    u0 = lax.bitcast_convert_type(
        t3[:, 0, :].astype(jnp.bfloat16), jnp.uint16).astype(jnp.uint32)
    u1 = lax.bitcast_convert_type(
        t3[:, 1, :].astype(jnp.bfloat16), jnp.uint16).astype(jnp.uint32)
    o_ref[...] = lax.bitcast_convert_type(u0 | (u1 << 16), jnp.int32)


def _pack(tables):
    tview = (tables.reshape(_N_LEVEL, _N_ENTRIES // 128, 128, 2)
             .transpose(0, 1, 3, 2).reshape(_N_LEVEL * _N_ENTRIES // 64, 128))
    R = 8192
    rows = tview.shape[0]
    packed = pl.pallas_call(
        _pack_body,
        grid=(rows // R,),
        in_specs=[pl.BlockSpec((R, 128), lambda i: (i, 0))],
        out_specs=pl.BlockSpec((R // 2, 128), lambda i: (i, 0)),
        out_shape=jax.ShapeDtypeStruct((rows // 2, 128), jnp.int32),
    )(tview)
    return packed.reshape(_N_LEVEL * _N_ENTRIES)


def _mlp_body(ft_ref, w1, b1, w2, b2, w3, b3, w4, b4, out_ref):
    dn = (((0,), (0,)), ((), ()))
    ft = ft_ref[...]
    h = jnp.maximum(
        lax.dot_general(w1[...], ft, dn, preferred_element_type=jnp.float32)
        + b1[...], 0.0)
    h = jnp.maximum(
        lax.dot_general(w2[...], h, dn, preferred_element_type=jnp.float32)
        + b2[...], 0.0)
    h = jnp.maximum(
        lax.dot_general(w3[...], h, dn, preferred_element_type=jnp.float32)
        + b3[...], 0.0)
    out_ref[...] = (
        lax.dot_general(h, w4[...], dn, preferred_element_type=jnp.float32)
        + b4[...])


def _mlp(featsT, W1, b1, W2, b2, W3, b3, W4, b4):
    BT = 8192
    grid = (_B // BT,)
    full = lambda shape: pl.BlockSpec(shape, lambda i: (0, 0))
    return pl.pallas_call(
        _mlp_body,
        grid=grid,
        in_specs=[
            pl.BlockSpec((32, BT), lambda i: (0, i)),
            full((32, 64)), full((64, 1)),
            full((64, 64)), full((64, 1)),
            full((64, 64)), full((64, 1)),
            full((64, 3)), full((1, 3)),
        ],
        out_specs=pl.BlockSpec((BT, 3), lambda i: (i, 0)),
        out_shape=jax.ShapeDtypeStruct((_B, 3), jnp.float32),
    )(featsT, W1, b1.reshape(64, 1), W2, b2.reshape(64, 1),
      W3, b3.reshape(64, 1), W4, b4.reshape(1, 3))


def kernel(x, tables, W1, b1, W2, b2, W3, b3, W4, b4):
    tbl = _pack(tables)
    featsT = _encode_sc(x.T.reshape(3 * _B), tbl).reshape(2 * _N_LEVEL, _B)
    return _mlp(featsT, W1, b1, W2, b2, W3, b3, W4, b4)
    return _mlp(featsT, W1, b1, W2, b2, W3, b3, W4, b4)
